# baseline (speedup 1.0000x reference)
import functools

import jax
import jax.numpy as jnp
from jax import lax
from jax.experimental import pallas as pl
from jax.experimental.pallas import tpu as pltpu
from jax.experimental.pallas import tpu_sc as plsc

N = 10000
NPAD = 10240
E = 320000
NC = 2
NS = 16
BLK = 128
PAD_ROWS = 64

EC = E // (NC * NS)
NB_DEG = -(-EC // BLK)
ECP = NB_DEG * BLK

ET = E // NS
ABLK = 64
CHUNK = 32
NCH = 10
NB_AGG = NCH * CHUNK
ETP = NB_AGG * ABLK

_MESH = plsc.VectorSubcoreMesh(
    core_axis_name="c", subcore_axis_name="s", num_cores=NC, num_subcores=NS)


@functools.partial(
    pl.kernel,
    out_type=jax.ShapeDtypeStruct((NC * NPAD,), jnp.float32),
    mesh=_MESH,
    scratch_types=[
        pltpu.VMEM((NB_DEG, BLK), jnp.int32),
        pltpu.VMEM((640,), jnp.float32),
        pltpu.VMEM((BLK,), jnp.float32),
        pltpu.VMEM_SHARED((NPAD,), jnp.float32),
    ],
)
def _deg_kernel(dstd_hbm, deg_hbm, idx_v, zbuf, ones_v, acc):
    c = lax.axis_index("c")
    s = lax.axis_index("s")
    t = c * NS + s
    for k in range(640 // 16):
        zbuf[pl.ds(k * 16, 16)] = jnp.zeros((16,), jnp.float32)
    for k in range(BLK // 16):
        ones_v[pl.ds(k * 16, 16)] = jnp.full((16,), 1.0, jnp.float32)

    pltpu.sync_copy(dstd_hbm.at[t], idx_v)
    pltpu.sync_copy(zbuf, acc.at[pl.ds(s * 640, 640)])
    plsc.subcore_barrier()

    def body(j, carry):
        pltpu.sync_copy(ones_v, acc.at[idx_v.at[j]], add=True)
        return carry

    lax.fori_loop(0, NB_DEG, body, 0)
    plsc.subcore_barrier()
    pltpu.sync_copy(acc.at[pl.ds(s * 640, 640)],
                    deg_hbm.at[pl.ds(c * NPAD + s * 640, 640)])


@functools.partial(
    pl.kernel,
    out_type=jax.ShapeDtypeStruct((NC * NPAD, 128), jnp.float32),
    mesh=_MESH,
    scratch_types=[
        pltpu.VMEM((CHUNK, ABLK), jnp.int32),
        pltpu.VMEM((CHUNK, ABLK), jnp.int32),
        pltpu.VMEM((4, ABLK, 128), jnp.float32),
        pltpu.VMEM_SHARED((NPAD + PAD_ROWS, 128), jnp.float32),
        pltpu.SemaphoreType.DMA((4,)),
    ],
)
def _agg_kernel(y_hbm, src_hbm, dst_hbm, out_hbm, sidx_v, didx_v, rows_v, acc,
                gsem):
    c = lax.axis_index("c")
    s = lax.axis_index("s")
    rows_per_tile = NPAD // NS
    src_t = src_hbm.at[c * NS + s]
    dst_t = dst_hbm.at[s]

    pltpu.sync_copy(
        y_hbm.at[pl.ds(c * NPAD + s * rows_per_tile, rows_per_tile)],
        acc.at[pl.ds(s * rows_per_tile, rows_per_tile)])
    plsc.subcore_barrier()

    def chunk(ch, carry):
        pltpu.sync_copy(src_t.at[ch], sidx_v)
        pltpu.sync_copy(dst_t.at[ch], didx_v)
        for k in range(3):
            pltpu.async_copy(y_hbm.at[sidx_v.at[k]], rows_v.at[k],
                             gsem.at[k])

        def body(j, carry2):
            p = lax.rem(j, 4)
            pltpu.make_async_copy(y_hbm.at[sidx_v.at[j]], rows_v.at[p],
                                  gsem.at[p]).wait()

            @pl.when(j + 3 < CHUNK)
            def _():
                q = lax.rem(j + 3, 4)
                pltpu.async_copy(y_hbm.at[sidx_v.at[j + 3]], rows_v.at[q],
                                 gsem.at[q])

            pltpu.sync_copy(rows_v.at[p], acc.at[didx_v.at[j]], add=True)
            return carry2

        lax.fori_loop(0, CHUNK, body, 0)
        return carry

    lax.fori_loop(0, NCH, chunk, 0)
    plsc.subcore_barrier()
    pltpu.sync_copy(
        acc.at[pl.ds(s * rows_per_tile, rows_per_tile)],
        out_hbm.at[pl.ds(c * NPAD + s * rows_per_tile, rows_per_tile)])


_R = 2048
_GRID = NPAD // _R


def _dis(d_ref):
    return lax.rsqrt(d_ref[0] + d_ref[1] + 1.0)


def _k1_body(x_ref, w_ref, d_ref, o_ref):
    y = jnp.dot(x_ref[...], w_ref[...],
                preferred_element_type=jnp.float32) * _dis(d_ref)
    o_ref[0] = y[:, :128]
    o_ref[1] = y[:, 128:]


def _k23_body(a_ref, w_ref, b_ref, d_ref, o_ref):
    dis = _dis(d_ref)
    h0 = jnp.maximum(a_ref[0] * dis + b_ref[:, :128], 0.0)
    h1 = jnp.maximum(a_ref[1] * dis + b_ref[:, 128:], 0.0)
    y = (jnp.dot(h0, w_ref[:128], preferred_element_type=jnp.float32)
         + jnp.dot(h1, w_ref[128:], preferred_element_type=jnp.float32)) * dis
    o_ref[0] = y[:, :128]
    o_ref[1] = y[:, 128:]


def _k4_body(a_ref, b_ref, d_ref, o_ref):
    dis = _dis(d_ref)
    o_ref[...] = jnp.concatenate(
        [a_ref[0] * dis + b_ref[:, :128], a_ref[1] * dis + b_ref[:, 128:]],
        axis=1)


_ysplit_shape = jax.ShapeDtypeStruct((2, NPAD, 128), jnp.float32)
_spec_a = pl.BlockSpec((2, _R, 128), lambda i: (0, i, 0))
_spec_d = pl.BlockSpec((2, _R, 1), lambda i: (0, i, 0))
_spec_b = pl.BlockSpec((1, 256), lambda i: (0, 0))

_k1 = pl.pallas_call(
    _k1_body, grid=(_GRID,),
    in_specs=[pl.BlockSpec((_R, 128), lambda i: (i, 0)),
              pl.BlockSpec((128, 256), lambda i: (0, 0)),
              _spec_d],
    out_specs=_spec_a,
    out_shape=_ysplit_shape)

_k23 = pl.pallas_call(
    _k23_body, grid=(_GRID,),
    in_specs=[_spec_a,
              pl.BlockSpec((256, 256), lambda i: (0, 0)),
              _spec_b,
              _spec_d],
    out_specs=_spec_a,
    out_shape=_ysplit_shape)

_k4 = pl.pallas_call(
    _k4_body, grid=(_GRID,),
    in_specs=[_spec_a, _spec_b, _spec_d],
    out_specs=pl.BlockSpec((_R, 256), lambda i: (i, 0)),
    out_shape=jax.ShapeDtypeStruct((N, 256), jnp.float32))


def kernel(x, edge_index, W1, b1, W2, b2, W3, b3):
    src = edge_index[0]
    dst = edge_index[1]

    padi_d = (jnp.arange(ECP - EC, dtype=jnp.int32) % PAD_ROWS)
    dstd = jnp.concatenate(
        [dst.reshape(NC * NS, EC),
         jnp.broadcast_to(N + padi_d, (NC * NS, ECP - EC))],
        axis=1).reshape(NC * NS, NB_DEG, BLK)

    padi_a = (jnp.arange(ETP - ET, dtype=jnp.int32) % PAD_ROWS)
    srcr = jnp.concatenate(
        [src.reshape(NS, ET), jnp.broadcast_to(padi_a, (NS, ETP - ET))],
        axis=1)
    dstr = jnp.concatenate(
        [dst.reshape(NS, ET),
         jnp.broadcast_to(NPAD + padi_a, (NS, ETP - ET))],
        axis=1).reshape(NS, NCH, CHUNK, ABLK)
    src2 = jnp.concatenate([srcr, srcr + NPAD],
                           axis=0).reshape(NC * NS, NCH, CHUNK, ABLK)

    deg = _deg_kernel(dstd).reshape(NC, NPAD, 1)

    y1 = _k1(x, W1, deg)
    a1 = _agg_kernel(y1.reshape(NC * NPAD, 128), src2, dstr)
    y2 = _k23(a1.reshape(2, NPAD, 128), W2, b1.reshape(1, 256), deg)
    a2 = _agg_kernel(y2.reshape(NC * NPAD, 128), src2, dstr)
    y3 = _k23(a2.reshape(2, NPAD, 128), W3, b2.reshape(1, 256), deg)
    a3 = _agg_kernel(y3.reshape(NC * NPAD, 128), src2, dstr)
    return _k4(a3.reshape(2, NPAD, 128), b3.reshape(1, 256), deg)


# speedup vs baseline: 1.0125x; 1.0125x over previous
import functools

import jax
import jax.numpy as jnp
from jax import lax
from jax.experimental import pallas as pl
from jax.experimental.pallas import tpu as pltpu
from jax.experimental.pallas import tpu_sc as plsc

N = 10000
NPAD = 10240
E = 320000
NC = 2
NS = 16
BLK = 128
PAD_ROWS = 64

EC = E // (NC * NS)
NB_DEG = -(-EC // BLK)
ECP = NB_DEG * BLK

ET = E // NS
ABLK = 64
CHUNK = 20
NCH = 16
NB_AGG = NCH * CHUNK
ETP = NB_AGG * ABLK

_MESH = plsc.VectorSubcoreMesh(
    core_axis_name="c", subcore_axis_name="s", num_cores=NC, num_subcores=NS)


@functools.partial(
    pl.kernel,
    out_type=jax.ShapeDtypeStruct((NC * NPAD,), jnp.float32),
    mesh=_MESH,
    scratch_types=[
        pltpu.VMEM((NB_DEG, BLK), jnp.int32),
        pltpu.VMEM((640,), jnp.float32),
        pltpu.VMEM((BLK,), jnp.float32),
        pltpu.VMEM_SHARED((NPAD,), jnp.float32),
    ],
)
def _deg_kernel(dstd_hbm, deg_hbm, idx_v, zbuf, ones_v, acc):
    c = lax.axis_index("c")
    s = lax.axis_index("s")
    t = c * NS + s
    for k in range(640 // 16):
        zbuf[pl.ds(k * 16, 16)] = jnp.zeros((16,), jnp.float32)
    for k in range(BLK // 16):
        ones_v[pl.ds(k * 16, 16)] = jnp.full((16,), 1.0, jnp.float32)

    pltpu.sync_copy(dstd_hbm.at[t], idx_v)
    pltpu.sync_copy(zbuf, acc.at[pl.ds(s * 640, 640)])
    plsc.subcore_barrier()

    def body(j, carry):
        pltpu.sync_copy(ones_v, acc.at[idx_v.at[j]], add=True)
        return carry

    lax.fori_loop(0, NB_DEG, body, 0)
    plsc.subcore_barrier()
    pltpu.sync_copy(acc.at[pl.ds(s * 640, 640)],
                    deg_hbm.at[pl.ds(c * NPAD + s * 640, 640)])


@functools.partial(
    pl.kernel,
    out_type=jax.ShapeDtypeStruct((NC * NPAD, 128), jnp.float32),
    mesh=_MESH,
    scratch_types=[
        pltpu.VMEM((2, CHUNK, ABLK), jnp.int32),
        pltpu.VMEM((2, CHUNK, ABLK), jnp.int32),
        pltpu.VMEM((4, ABLK, 128), jnp.float32),
        pltpu.VMEM_SHARED((NPAD, 128), jnp.float32),
        pltpu.SemaphoreType.DMA((4,)),
        pltpu.SemaphoreType.DMA,
    ],
)
def _agg_kernel(y_hbm, src_hbm, dst_hbm, out_hbm, sidx_v, didx_v, rows_v, acc,
                gsem, isem):
    c = lax.axis_index("c")
    s = lax.axis_index("s")
    rows_per_tile = NPAD // NS
    src_t = src_hbm.at[c * NS + s]
    dst_t = dst_hbm.at[s]

    pltpu.sync_copy(
        y_hbm.at[pl.ds(c * NPAD + s * rows_per_tile, rows_per_tile)],
        acc.at[pl.ds(s * rows_per_tile, rows_per_tile)])
    plsc.subcore_barrier()

    pltpu.async_copy(src_t.at[0], sidx_v.at[0], isem)
    pltpu.async_copy(dst_t.at[0], didx_v.at[0], isem)

    def chunk(ch, carry):
        p = lax.rem(ch, 2)
        pltpu.make_async_copy(src_t.at[ch], sidx_v.at[p], isem).wait()
        pltpu.make_async_copy(dst_t.at[ch], didx_v.at[p], isem).wait()

        @pl.when(ch + 1 < NCH)
        def _():
            q = 1 - p
            pltpu.async_copy(src_t.at[ch + 1], sidx_v.at[q], isem)
            pltpu.async_copy(dst_t.at[ch + 1], didx_v.at[q], isem)

        sidx_c = sidx_v.at[p]
        didx_c = didx_v.at[p]
        for k in range(3):
            pltpu.async_copy(y_hbm.at[sidx_c.at[k]], rows_v.at[k],
                             gsem.at[k])

        def body(j, carry2):
            pp = lax.rem(j, 4)
            pltpu.make_async_copy(y_hbm.at[sidx_c.at[j]], rows_v.at[pp],
                                  gsem.at[pp]).wait()

            @pl.when(j + 3 < CHUNK)
            def _():
                qq = lax.rem(j + 3, 4)
                pltpu.async_copy(y_hbm.at[sidx_c.at[j + 3]], rows_v.at[qq],
                                 gsem.at[qq])

            pltpu.sync_copy(rows_v.at[pp], acc.at[didx_c.at[j]], add=True)
            return carry2

        lax.fori_loop(0, CHUNK, body, 0)
        return carry

    lax.fori_loop(0, NCH, chunk, 0)
    plsc.subcore_barrier()
    pltpu.sync_copy(
        acc.at[pl.ds(s * rows_per_tile, rows_per_tile)],
        out_hbm.at[pl.ds(c * NPAD + s * rows_per_tile, rows_per_tile)])


_R = 2048
_GRID = NPAD // _R


def _dis(d_ref):
    return lax.rsqrt(d_ref[0] + d_ref[1] + 1.0)


def _k1_body(x_ref, w_ref, d_ref, o_ref):
    y = jnp.dot(x_ref[...], w_ref[...],
                preferred_element_type=jnp.float32) * _dis(d_ref)
    o_ref[0] = y[:, :128]
    o_ref[1] = y[:, 128:]


def _k23_body(a_ref, w_ref, b_ref, d_ref, o_ref):
    dis = _dis(d_ref)
    h0 = jnp.maximum(a_ref[0] * dis + b_ref[:, :128], 0.0)
    h1 = jnp.maximum(a_ref[1] * dis + b_ref[:, 128:], 0.0)
    y = (jnp.dot(h0, w_ref[:128], preferred_element_type=jnp.float32)
         + jnp.dot(h1, w_ref[128:], preferred_element_type=jnp.float32)) * dis
    o_ref[0] = y[:, :128]
    o_ref[1] = y[:, 128:]


def _k4_body(a_ref, b_ref, d_ref, o_ref):
    dis = _dis(d_ref)
    o_ref[...] = jnp.concatenate(
        [a_ref[0] * dis + b_ref[:, :128], a_ref[1] * dis + b_ref[:, 128:]],
        axis=1)


_ysplit_shape = jax.ShapeDtypeStruct((2, NPAD, 128), jnp.float32)
_spec_a = pl.BlockSpec((2, _R, 128), lambda i: (0, i, 0))
_spec_d = pl.BlockSpec((2, _R, 1), lambda i: (0, i, 0))
_spec_b = pl.BlockSpec((1, 256), lambda i: (0, 0))

_k1 = pl.pallas_call(
    _k1_body, grid=(_GRID,),
    in_specs=[pl.BlockSpec((_R, 128), lambda i: (i, 0)),
              pl.BlockSpec((128, 256), lambda i: (0, 0)),
              _spec_d],
    out_specs=_spec_a,
    out_shape=_ysplit_shape)

_k23 = pl.pallas_call(
    _k23_body, grid=(_GRID,),
    in_specs=[_spec_a,
              pl.BlockSpec((256, 256), lambda i: (0, 0)),
              _spec_b,
              _spec_d],
    out_specs=_spec_a,
    out_shape=_ysplit_shape)

_k4 = pl.pallas_call(
    _k4_body, grid=(_GRID,),
    in_specs=[_spec_a, _spec_b, _spec_d],
    out_specs=pl.BlockSpec((_R, 256), lambda i: (i, 0)),
    out_shape=jax.ShapeDtypeStruct((N, 256), jnp.float32))


def kernel(x, edge_index, W1, b1, W2, b2, W3, b3):
    src = edge_index[0]
    dst = edge_index[1]

    padi_d = (jnp.arange(ECP - EC, dtype=jnp.int32) % PAD_ROWS)
    dstd = jnp.concatenate(
        [dst.reshape(NC * NS, EC),
         jnp.broadcast_to(N + padi_d, (NC * NS, ECP - EC))],
        axis=1).reshape(NC * NS, NB_DEG, BLK)

    padi_a = (jnp.arange(ETP - ET, dtype=jnp.int32) % PAD_ROWS)
    srcr = jnp.concatenate(
        [src.reshape(NS, ET), jnp.broadcast_to(padi_a, (NS, ETP - ET))],
        axis=1)
    dstr = jnp.concatenate(
        [dst.reshape(NS, ET),
         jnp.broadcast_to(N + padi_a, (NS, ETP - ET))],
        axis=1).reshape(NS, NCH, CHUNK, ABLK)
    src2 = jnp.concatenate([srcr, srcr + NPAD],
                           axis=0).reshape(NC * NS, NCH, CHUNK, ABLK)

    deg = _deg_kernel(dstd).reshape(NC, NPAD, 1)

    y1 = _k1(x, W1, deg)
    a1 = _agg_kernel(y1.reshape(NC * NPAD, 128), src2, dstr)
    y2 = _k23(a1.reshape(2, NPAD, 128), W2, b1.reshape(1, 256), deg)
    a2 = _agg_kernel(y2.reshape(NC * NPAD, 128), src2, dstr)
    y3 = _k23(a2.reshape(2, NPAD, 128), W3, b2.reshape(1, 256), deg)
    a3 = _agg_kernel(y3.reshape(NC * NPAD, 128), src2, dstr)
    return _k4(a3.reshape(2, NPAD, 128), b3.reshape(1, 256), deg)
